# baseline (speedup 1.0000x reference)
import functools

import jax
import jax.numpy as jnp
from jax import lax
from jax.experimental import pallas as pl
from jax.experimental.pallas import tpu as pltpu
from jax.experimental.pallas import tpu_sc as plsc

N = 10000
E = 160000
D = 256
HD = 128
NC = 2
NS = 16
EPT = E // NS
K = 80
NCHUNK = EPT // K
PHASES = 5
CPP = NCHUNK // PHASES
NPAD = 10240
RPT = NPAD // NS
ZROWS = 32
DPT = NPAD // NS


def _sc_agg_body(with_deg, x_hbm, src_hbm, dst_hbm, summed_hbm, deg_hbm,
                 src0, src1, dst0, dst1, rows0, rows1, rows2, ones_v, zdeg,
                 acc, degacc, sem0, sem1, sem2, ssem0, ssem1, ssem2, semi):
    c = lax.axis_index("c")
    s = lax.axis_index("s")
    zero16 = jnp.zeros((16,), jnp.float32)
    one16 = jnp.ones((16,), jnp.float32)

    def _zrow(t, carry):
        i = t // (HD // 16)
        k = t % (HD // 16)
        rows0[i, pl.ds(k * 16, 16)] = zero16
        return carry
    lax.fori_loop(0, K * (HD // 16), _zrow, None)

    def _zdeg(t, carry):
        zdeg[pl.ds(t * 16, 16)] = zero16
        return carry
    lax.fori_loop(0, DPT // 16, _zdeg, None)

    def _ones(t, carry):
        ones_v[pl.ds(t * 16, 16)] = one16
        return carry
    lax.fori_loop(0, K // 16, _ones, None)

    for k in range(RPT // K):
        pltpu.sync_copy(rows0, acc.at[pl.ds(s * RPT + k * K, K)])

    if with_deg:
        @pl.when(c == 0)
        def _():
            pltpu.sync_copy(zdeg, degacc.at[pl.ds(s * DPT, DPT)])

    plsc.subcore_barrier()

    sbufs = (src0, src1)
    dbufs = (dst0, dst1)

    pltpu.sync_copy(src_hbm.at[s, 0], src0)
    pltpu.sync_copy(dst_hbm.at[s, 0], dst0)

    for p in range(PHASES):
        sv = sbufs[p % 2]
        dv = dbufs[p % 2]
        if p + 1 < PHASES:
            pltpu.async_copy(src_hbm.at[s, p + 1], sbufs[(p + 1) % 2], semi)
            pltpu.async_copy(dst_hbm.at[s, p + 1], dbufs[(p + 1) % 2], semi)

        bufs = ((rows0, sem0, ssem0), (rows1, sem1, ssem1),
                (rows2, sem2, ssem2))

        x_core = x_hbm.at[:, c]

        def _start_gather(j, buf, sem, sv=sv):
            pltpu.async_copy(x_core.at[sv.at[j]], buf, sem)

        def _finish(j, buf, gsem, ssem, sv=sv, dv=dv):
            pltpu.make_async_copy(x_core.at[sv.at[j]], buf, gsem).wait()
            pltpu.async_copy(buf, acc.at[dv.at[j]], ssem, add=True)

            if with_deg:
                @pl.when(c == 0)
                def _():
                    pltpu.sync_copy(ones_v, degacc.at[dv.at[j]], add=True)

        _start_gather(0, rows0, sem0)
        _start_gather(1, rows1, sem1)

        def _trip(t, carry):
            j0 = 3 * t
            for u in range(3):
                j = j0 + u
                bn, gsn, ssn = bufs[(u + 2) % 3]

                @pl.when(j + 2 < CPP)
                def _(j=j, bn=bn, gsn=gsn, ssn=ssn):
                    @pl.when(j >= 1)
                    def _():
                        pltpu.make_async_copy(
                            bn, acc.at[dv.at[j - 1]], ssn).wait()
                    _start_gather(j + 2, bn, gsn)
                _finish(j, *bufs[u])
            return carry
        lax.fori_loop(0, CPP // 3, _trip, None)
        _finish(CPP - 1, *bufs[0])
        for jj, u in ((CPP - 3, 1), (CPP - 2, 2), (CPP - 1, 0)):
            pltpu.make_async_copy(bufs[u][0], acc.at[dv.at[jj]],
                                  bufs[u][2]).wait()

        if p + 1 < PHASES:
            pltpu.make_async_copy(src_hbm.at[s, p + 1],
                                  sbufs[(p + 1) % 2], semi).wait()
            pltpu.make_async_copy(dst_hbm.at[s, p + 1],
                                  dbufs[(p + 1) % 2], semi).wait()

    plsc.subcore_barrier()

    pltpu.sync_copy(acc.at[pl.ds(s * RPT, RPT)],
                    summed_hbm.at[c, pl.ds(s * RPT, RPT)])

    if with_deg:
        @pl.when(c == 0)
        def _():
            pltpu.sync_copy(degacc.at[pl.ds(s * DPT, DPT)],
                            deg_hbm.at[pl.ds(s * DPT, DPT)])


def _make_sc_aggregate(with_deg):
    return functools.partial(
        pl.kernel,
        out_type=[jax.ShapeDtypeStruct((NC, NPAD, HD), jnp.float32),
                  jax.ShapeDtypeStruct((NPAD,), jnp.float32)],
        mesh=plsc.VectorSubcoreMesh(core_axis_name="c", subcore_axis_name="s"),
        scratch_types=[
            pltpu.VMEM((CPP, K), jnp.int32),
            pltpu.VMEM((CPP, K), jnp.int32),
            pltpu.VMEM((CPP, K), jnp.int32),
            pltpu.VMEM((CPP, K), jnp.int32),
            pltpu.VMEM((K, HD), jnp.float32),
            pltpu.VMEM((K, HD), jnp.float32),
            pltpu.VMEM((K, HD), jnp.float32),
            pltpu.VMEM((K,), jnp.float32),
            pltpu.VMEM((DPT,), jnp.float32),
            pltpu.VMEM_SHARED((NPAD, HD), jnp.float32),
            pltpu.VMEM_SHARED((NPAD,), jnp.float32),
            pltpu.SemaphoreType.DMA,
            pltpu.SemaphoreType.DMA,
            pltpu.SemaphoreType.DMA,
            pltpu.SemaphoreType.DMA,
            pltpu.SemaphoreType.DMA,
            pltpu.SemaphoreType.DMA,
            pltpu.SemaphoreType.DMA,
        ],
    )(functools.partial(_sc_agg_body, with_deg))


_sc_aggregate_deg = _make_sc_aggregate(True)
_sc_aggregate = _make_sc_aggregate(False)


def _tc_skip_body(in_split, xref, wrref, bref, oref):
    if in_split:
        xx = jnp.concatenate([xref[0], xref[1]], axis=-1)
    else:
        xx = xref[...]
    oref[...] = (jnp.dot(xx, wrref[...], preferred_element_type=jnp.float32)
                 + bref[...])


def _tc_skip(xin, W_r, b, *, in_split):
    BM = 2000
    grid = (N // BM,)
    split_spec = pl.BlockSpec((NC, BM, HD), lambda i: (0, i, 0))
    dense_spec = pl.BlockSpec((BM, D), lambda i: (i, 0))
    return pl.pallas_call(
        functools.partial(_tc_skip_body, in_split),
        grid=grid,
        in_specs=[split_spec if in_split else dense_spec,
                  pl.BlockSpec((D, D), lambda i: (0, 0)),
                  pl.BlockSpec((1, D), lambda i: (0, 0))],
        out_specs=dense_spec,
        out_shape=jax.ShapeDtypeStruct((N, D), jnp.float32),
    )(xin, W_r, b.reshape(1, D))


def _tc_fused1_body(sref, dref, kref, wlref, w2rref, b2ref,
                    href, k2ref):
    agg = jnp.concatenate([sref[0], sref[1]], axis=-1)
    rec = 1.0 / jnp.maximum(dref[...], 1.0)
    agg = agg * rec
    h = jnp.maximum(
        jnp.dot(agg, wlref[...], preferred_element_type=jnp.float32)
        + kref[...], 0.0)
    href[...] = h.reshape(href.shape)
    k2ref[...] = (jnp.dot(h, w2rref[...], preferred_element_type=jnp.float32)
                  + b2ref[...])


def _tc_fused1(summed, deg_col, skip, W_l, W2_r, b2):
    BM = 2000
    grid = (N // BM,)
    split_spec = pl.BlockSpec((NC, BM, HD), lambda i: (0, i, 0))
    dense_spec = pl.BlockSpec((BM, D), lambda i: (i, 0))
    return pl.pallas_call(
        _tc_fused1_body,
        grid=grid,
        in_specs=[
            split_spec,
            pl.BlockSpec((BM, 1), lambda i: (i, 0)),
            dense_spec,
            pl.BlockSpec((D, D), lambda i: (0, 0)),
            pl.BlockSpec((D, D), lambda i: (0, 0)),
            pl.BlockSpec((1, D), lambda i: (0, 0)),
        ],
        out_specs=[pl.BlockSpec((BM, NC, HD), lambda i: (i, 0, 0)),
                   dense_spec],
        out_shape=[jax.ShapeDtypeStruct((N, NC, HD), jnp.float32),
                   jax.ShapeDtypeStruct((N, D), jnp.float32)],
    )(summed, deg_col, skip, W_l, W2_r, b2.reshape(1, D))


def _tc_main_body(sref, dref, kref, wlref, oref):
    agg = jnp.concatenate([sref[0], sref[1]], axis=-1)
    rec = 1.0 / jnp.maximum(dref[...], 1.0)
    agg = agg * rec
    oref[...] = (jnp.dot(agg, wlref[...], preferred_element_type=jnp.float32)
                 + kref[...])


def _tc_main(summed, deg_col, skip, W_l):
    BM = 2000
    grid = (N // BM,)
    split_spec = pl.BlockSpec((NC, BM, HD), lambda i: (0, i, 0))
    dense_spec = pl.BlockSpec((BM, D), lambda i: (i, 0))
    return pl.pallas_call(
        _tc_main_body,
        grid=grid,
        in_specs=[
            split_spec,
            pl.BlockSpec((BM, 1), lambda i: (i, 0)),
            dense_spec,
            pl.BlockSpec((D, D), lambda i: (0, 0)),
        ],
        out_specs=dense_spec,
        out_shape=jax.ShapeDtypeStruct((N, D), jnp.float32),
    )(summed, deg_col, skip, W_l)


def kernel(x, edge_index, W1_l, b1, W1_r, W2_l, b2, W2_r):
    src = edge_index[0].astype(jnp.int32)
    dst = edge_index[1].astype(jnp.int32)
    src_idx = src.reshape(NS, PHASES, CPP, K)
    dst_idx = dst.reshape(NS, PHASES, CPP, K)
    x3 = x.reshape(N, NC, HD)

    summed1, deg_pad = _sc_aggregate_deg(x3, src_idx, dst_idx)
    skip1 = _tc_skip(x, W1_r, b1, in_split=False)
    deg_col = deg_pad[:N].reshape(N, 1)
    h3, skip2 = _tc_fused1(summed1, deg_col, skip1, W1_l, W2_r, b2)
    summed2, _ = _sc_aggregate(h3, src_idx, dst_idx)
    out = _tc_main(summed2, deg_col, skip2, W2_l)
    return out


# speedup vs baseline: 1.0823x; 1.0823x over previous
import functools

import jax
import jax.numpy as jnp
from jax import lax
from jax.experimental import pallas as pl
from jax.experimental.pallas import tpu as pltpu
from jax.experimental.pallas import tpu_sc as plsc

N = 10000
E = 160000
D = 256
HD = 128
NC = 2
NS = 16
EPT = E // NS
K = 80
NCHUNK = EPT // K
PHASES = 5
CPP = NCHUNK // PHASES
NPAD = 10240
RPT = NPAD // NS
ZROWS = 32
DPT = NPAD // NS


def _sc_agg_body(with_deg, x_hbm, src_hbm, dst_hbm, summed_hbm, deg_hbm,
                 src0, src1, dst0, dst1, rows0, rows1, rows2, ones_v, zdeg,
                 acc, degacc, sem0, sem1, sem2, ssem0, ssem1, ssem2, semi):
    c = lax.axis_index("c")
    s = lax.axis_index("s")
    zero16 = jnp.zeros((16,), jnp.float32)
    one16 = jnp.ones((16,), jnp.float32)

    def _zrow(t, carry):
        i = t // (HD // 16)
        k = t % (HD // 16)
        rows0[i, pl.ds(k * 16, 16)] = zero16
        return carry
    lax.fori_loop(0, K * (HD // 16), _zrow, None)

    def _zdeg(t, carry):
        zdeg[pl.ds(t * 16, 16)] = zero16
        return carry
    lax.fori_loop(0, DPT // 16, _zdeg, None)

    def _ones(t, carry):
        ones_v[pl.ds(t * 16, 16)] = one16
        return carry
    lax.fori_loop(0, K // 16, _ones, None)

    for k in range(RPT // K):
        pltpu.sync_copy(rows0, acc.at[pl.ds(s * RPT + k * K, K)])

    if with_deg:
        @pl.when(c == 0)
        def _():
            pltpu.sync_copy(zdeg, degacc.at[pl.ds(s * DPT, DPT)])

    plsc.subcore_barrier()

    sbufs = (src0, src1)
    dbufs = (dst0, dst1)

    pltpu.sync_copy(src_hbm.at[c, s, 0], src0)
    pltpu.sync_copy(dst_hbm.at[s, 0], dst0)

    for p in range(PHASES):
        sv = sbufs[p % 2]
        dv = dbufs[p % 2]
        if p + 1 < PHASES:
            pltpu.async_copy(src_hbm.at[c, s, p + 1], sbufs[(p + 1) % 2], semi)
            pltpu.async_copy(dst_hbm.at[s, p + 1], dbufs[(p + 1) % 2], semi)

        bufs = ((rows0, sem0, ssem0), (rows1, sem1, ssem1),
                (rows2, sem2, ssem2))

        def _start_gather(j, buf, sem, sv=sv):
            pltpu.async_copy(x_hbm.at[sv.at[j]], buf, sem)

        def _finish(j, buf, gsem, ssem, sv=sv, dv=dv):
            pltpu.make_async_copy(x_hbm.at[sv.at[j]], buf, gsem).wait()
            pltpu.async_copy(buf, acc.at[dv.at[j]], ssem, add=True)

            if with_deg:
                @pl.when(c == 0)
                def _():
                    pltpu.sync_copy(ones_v, degacc.at[dv.at[j]], add=True)

        _start_gather(0, rows0, sem0)
        _start_gather(1, rows1, sem1)

        def _trip(t, carry):
            j0 = 3 * t
            for u in range(3):
                j = j0 + u
                bn, gsn, ssn = bufs[(u + 2) % 3]

                @pl.when(j + 2 < CPP)
                def _(j=j, bn=bn, gsn=gsn, ssn=ssn):
                    @pl.when(j >= 1)
                    def _():
                        pltpu.make_async_copy(
                            bn, acc.at[dv.at[j - 1]], ssn).wait()
                    _start_gather(j + 2, bn, gsn)
                _finish(j, *bufs[u])
            return carry
        lax.fori_loop(0, CPP // 3, _trip, None)
        _finish(CPP - 1, *bufs[0])
        for jj, u in ((CPP - 3, 1), (CPP - 2, 2), (CPP - 1, 0)):
            pltpu.make_async_copy(bufs[u][0], acc.at[dv.at[jj]],
                                  bufs[u][2]).wait()

        if p + 1 < PHASES:
            pltpu.make_async_copy(src_hbm.at[c, s, p + 1],
                                  sbufs[(p + 1) % 2], semi).wait()
            pltpu.make_async_copy(dst_hbm.at[s, p + 1],
                                  dbufs[(p + 1) % 2], semi).wait()

    plsc.subcore_barrier()

    pltpu.sync_copy(acc.at[pl.ds(s * RPT, RPT)],
                    summed_hbm.at[c, pl.ds(s * RPT, RPT)])

    if with_deg:
        @pl.when(c == 0)
        def _():
            pltpu.sync_copy(degacc.at[pl.ds(s * DPT, DPT)],
                            deg_hbm.at[pl.ds(s * DPT, DPT)])


def _make_sc_aggregate(with_deg):
    return functools.partial(
        pl.kernel,
        out_type=[jax.ShapeDtypeStruct((NC, NPAD, HD), jnp.float32),
                  jax.ShapeDtypeStruct((NPAD,), jnp.float32)],
        mesh=plsc.VectorSubcoreMesh(core_axis_name="c", subcore_axis_name="s"),
        scratch_types=[
            pltpu.VMEM((CPP, K), jnp.int32),
            pltpu.VMEM((CPP, K), jnp.int32),
            pltpu.VMEM((CPP, K), jnp.int32),
            pltpu.VMEM((CPP, K), jnp.int32),
            pltpu.VMEM((K, HD), jnp.float32),
            pltpu.VMEM((K, HD), jnp.float32),
            pltpu.VMEM((K, HD), jnp.float32),
            pltpu.VMEM((K,), jnp.float32),
            pltpu.VMEM((DPT,), jnp.float32),
            pltpu.VMEM_SHARED((NPAD, HD), jnp.float32),
            pltpu.VMEM_SHARED((NPAD,), jnp.float32),
            pltpu.SemaphoreType.DMA,
            pltpu.SemaphoreType.DMA,
            pltpu.SemaphoreType.DMA,
            pltpu.SemaphoreType.DMA,
            pltpu.SemaphoreType.DMA,
            pltpu.SemaphoreType.DMA,
            pltpu.SemaphoreType.DMA,
        ],
    )(functools.partial(_sc_agg_body, with_deg))


_sc_aggregate_deg = _make_sc_aggregate(True)
_sc_aggregate = _make_sc_aggregate(False)


def _tc_skip_body(in_split, xref, wrref, bref, oref):
    if in_split:
        xx = jnp.concatenate([xref[0], xref[1]], axis=-1)
    else:
        xx = xref[...]
    oref[...] = (jnp.dot(xx, wrref[...], preferred_element_type=jnp.float32)
                 + bref[...])


def _tc_skip(xin, W_r, b, *, in_split):
    BM = 2000
    grid = (N // BM,)
    split_spec = pl.BlockSpec((NC, BM, HD), lambda i: (0, i, 0))
    dense_spec = pl.BlockSpec((BM, D), lambda i: (i, 0))
    return pl.pallas_call(
        functools.partial(_tc_skip_body, in_split),
        grid=grid,
        in_specs=[split_spec if in_split else dense_spec,
                  pl.BlockSpec((D, D), lambda i: (0, 0)),
                  pl.BlockSpec((1, D), lambda i: (0, 0))],
        out_specs=dense_spec,
        out_shape=jax.ShapeDtypeStruct((N, D), jnp.float32),
    )(xin, W_r, b.reshape(1, D))


def _tc_fused1_body(sref, dref, kref, wlref, w2rref, b2ref,
                    href, k2ref):
    agg = jnp.concatenate([sref[0], sref[1]], axis=-1)
    rec = 1.0 / jnp.maximum(dref[...], 1.0)
    agg = agg * rec
    h = jnp.maximum(
        jnp.dot(agg, wlref[...], preferred_element_type=jnp.float32)
        + kref[...], 0.0)
    href[0] = h[:, :HD]
    href[1] = h[:, HD:]
    k2ref[...] = (jnp.dot(h, w2rref[...], preferred_element_type=jnp.float32)
                  + b2ref[...])


def _tc_fused1(summed, deg_col, skip, W_l, W2_r, b2):
    BM = 2000
    grid = (N // BM,)
    split_spec = pl.BlockSpec((NC, BM, HD), lambda i: (0, i, 0))
    dense_spec = pl.BlockSpec((BM, D), lambda i: (i, 0))
    return pl.pallas_call(
        _tc_fused1_body,
        grid=grid,
        in_specs=[
            split_spec,
            pl.BlockSpec((BM, 1), lambda i: (i, 0)),
            dense_spec,
            pl.BlockSpec((D, D), lambda i: (0, 0)),
            pl.BlockSpec((D, D), lambda i: (0, 0)),
            pl.BlockSpec((1, D), lambda i: (0, 0)),
        ],
        out_specs=[split_spec, dense_spec],
        out_shape=[jax.ShapeDtypeStruct((NC, N, HD), jnp.float32),
                   jax.ShapeDtypeStruct((N, D), jnp.float32)],
    )(summed, deg_col, skip, W_l, W2_r, b2.reshape(1, D))


def _tc_main_body(sref, dref, kref, wlref, oref):
    agg = jnp.concatenate([sref[0], sref[1]], axis=-1)
    rec = 1.0 / jnp.maximum(dref[...], 1.0)
    agg = agg * rec
    oref[...] = (jnp.dot(agg, wlref[...], preferred_element_type=jnp.float32)
                 + kref[...])


def _tc_main(summed, deg_col, skip, W_l):
    BM = 2000
    grid = (N // BM,)
    split_spec = pl.BlockSpec((NC, BM, HD), lambda i: (0, i, 0))
    dense_spec = pl.BlockSpec((BM, D), lambda i: (i, 0))
    return pl.pallas_call(
        _tc_main_body,
        grid=grid,
        in_specs=[
            split_spec,
            pl.BlockSpec((BM, 1), lambda i: (i, 0)),
            dense_spec,
            pl.BlockSpec((D, D), lambda i: (0, 0)),
        ],
        out_specs=dense_spec,
        out_shape=jax.ShapeDtypeStruct((N, D), jnp.float32),
    )(summed, deg_col, skip, W_l)


def kernel(x, edge_index, W1_l, b1, W1_r, W2_l, b2, W2_r):
    src = edge_index[0].astype(jnp.int32)
    dst = edge_index[1].astype(jnp.int32)
    sr = src.reshape(NS, PHASES, CPP, K)
    src_idx = jnp.stack([sr, sr + N])
    dst_idx = dst.reshape(NS, PHASES, CPP, K)
    x_flat = jnp.concatenate([x[:, :HD], x[:, HD:]], axis=0)

    summed1, deg_pad = _sc_aggregate_deg(x_flat, src_idx, dst_idx)
    skip1 = _tc_skip(x, W1_r, b1, in_split=False)
    deg_col = deg_pad[:N].reshape(N, 1)
    h_split, skip2 = _tc_fused1(summed1, deg_col, skip1, W1_l, W2_r, b2)
    summed2, _ = _sc_aggregate(h_split.reshape(NC * N, HD), src_idx, dst_idx)
    out = _tc_main(summed2, deg_col, skip2, W2_l)
    return out


# speedup vs baseline: 1.0921x; 1.0090x over previous
import functools

import jax
import jax.numpy as jnp
from jax import lax
from jax.experimental import pallas as pl
from jax.experimental.pallas import tpu as pltpu
from jax.experimental.pallas import tpu_sc as plsc

N = 10000
E = 160000
D = 256
HD = 128
NC = 2
NS = 16
EPT = E // NS
K = 80
NCHUNK = EPT // K
PHASES = 5
CPP = NCHUNK // PHASES
NPAD = 10240
RPT = NPAD // NS
ZROWS = 32
DPT = NPAD // NS


def _sc_agg_body(with_deg, x_hbm, src_hbm, dst_hbm, summed_hbm, deg_hbm,
                 src0, src1, dst0, dst1, rows0, rows1, rows2, ones_v, zdeg,
                 acc, degacc, sem0, sem1, sem2, ssem0, ssem1, ssem2, semi):
    c = lax.axis_index("c")
    s = lax.axis_index("s")
    zero16 = jnp.zeros((16,), jnp.float32)
    one16 = jnp.ones((16,), jnp.float32)

    def _zrow(t, carry):
        i = t // (HD // 16)
        k = t % (HD // 16)
        rows0[i, pl.ds(k * 16, 16)] = zero16
        return carry
    lax.fori_loop(0, K * (HD // 16), _zrow, None)

    def _zdeg(t, carry):
        zdeg[pl.ds(t * 16, 16)] = zero16
        return carry
    lax.fori_loop(0, DPT // 16, _zdeg, None)

    def _ones(t, carry):
        ones_v[pl.ds(t * 16, 16)] = one16
        return carry
    lax.fori_loop(0, K // 16, _ones, None)

    for k in range(RPT // K):
        pltpu.sync_copy(rows0, acc.at[pl.ds(s * RPT + k * K, K)])

    if with_deg:
        @pl.when(c == 0)
        def _():
            pltpu.sync_copy(zdeg, degacc.at[pl.ds(s * DPT, DPT)])

    plsc.subcore_barrier()

    sbufs = (src0, src1)
    dbufs = (dst0, dst1)

    pltpu.sync_copy(src_hbm.at[c, s, 0], src0)
    pltpu.sync_copy(dst_hbm.at[s, 0], dst0)

    for p in range(PHASES):
        sv = sbufs[p % 2]
        dv = dbufs[p % 2]
        if p + 1 < PHASES:
            pltpu.async_copy(src_hbm.at[c, s, p + 1], sbufs[(p + 1) % 2], semi)
            pltpu.async_copy(dst_hbm.at[s, p + 1], dbufs[(p + 1) % 2], semi)

        bufs = ((rows0, sem0, ssem0), (rows1, sem1, ssem1),
                (rows2, sem2, ssem2))

        def _start_gather(j, buf, sem, sv=sv):
            pltpu.async_copy(x_hbm.at[sv.at[j]], buf, sem)

        def _finish(j, buf, gsem, ssem, sv=sv, dv=dv):
            pltpu.make_async_copy(x_hbm.at[sv.at[j]], buf, gsem).wait()
            pltpu.async_copy(buf, acc.at[dv.at[j]], ssem, add=True)

            if with_deg:
                @pl.when(c == 0)
                def _():
                    pltpu.sync_copy(ones_v, degacc.at[dv.at[j]], add=True)

        _start_gather(0, rows0, sem0)
        _start_gather(1, rows1, sem1)

        def _trip(t, carry):
            j0 = 3 * t
            for u in range(3):
                j = j0 + u
                bn, gsn, ssn = bufs[(u + 2) % 3]

                @pl.when(j + 2 < CPP)
                def _(j=j, bn=bn, gsn=gsn, ssn=ssn):
                    @pl.when(j >= 1)
                    def _():
                        pltpu.make_async_copy(
                            bn, acc.at[dv.at[j - 1]], ssn).wait()
                    _start_gather(j + 2, bn, gsn)
                _finish(j, *bufs[u])
            return carry
        lax.fori_loop(0, CPP // 3, _trip, None)
        _finish(CPP - 1, *bufs[0])
        for jj, u in ((CPP - 3, 1), (CPP - 2, 2), (CPP - 1, 0)):
            pltpu.make_async_copy(bufs[u][0], acc.at[dv.at[jj]],
                                  bufs[u][2]).wait()

        if p + 1 < PHASES:
            pltpu.make_async_copy(src_hbm.at[c, s, p + 1],
                                  sbufs[(p + 1) % 2], semi).wait()
            pltpu.make_async_copy(dst_hbm.at[s, p + 1],
                                  dbufs[(p + 1) % 2], semi).wait()

    plsc.subcore_barrier()

    pltpu.sync_copy(acc.at[pl.ds(s * RPT, RPT)],
                    summed_hbm.at[c, pl.ds(s * RPT, RPT)])

    if with_deg:
        @pl.when(c == 0)
        def _():
            pltpu.sync_copy(degacc.at[pl.ds(s * DPT, DPT)],
                            deg_hbm.at[pl.ds(s * DPT, DPT)])


def _make_sc_aggregate(with_deg):
    return functools.partial(
        pl.kernel,
        out_type=[jax.ShapeDtypeStruct((NC, NPAD, HD), jnp.float32),
                  jax.ShapeDtypeStruct((NPAD,), jnp.float32)],
        mesh=plsc.VectorSubcoreMesh(core_axis_name="c", subcore_axis_name="s"),
        scratch_types=[
            pltpu.VMEM((CPP, K), jnp.int32),
            pltpu.VMEM((CPP, K), jnp.int32),
            pltpu.VMEM((CPP, K), jnp.int32),
            pltpu.VMEM((CPP, K), jnp.int32),
            pltpu.VMEM((K, HD), jnp.float32),
            pltpu.VMEM((K, HD), jnp.float32),
            pltpu.VMEM((K, HD), jnp.float32),
            pltpu.VMEM((K,), jnp.float32),
            pltpu.VMEM((DPT,), jnp.float32),
            pltpu.VMEM_SHARED((NPAD, HD), jnp.float32),
            pltpu.VMEM_SHARED((NPAD,), jnp.float32),
            pltpu.SemaphoreType.DMA,
            pltpu.SemaphoreType.DMA,
            pltpu.SemaphoreType.DMA,
            pltpu.SemaphoreType.DMA,
            pltpu.SemaphoreType.DMA,
            pltpu.SemaphoreType.DMA,
            pltpu.SemaphoreType.DMA,
        ],
    )(functools.partial(_sc_agg_body, with_deg))


_sc_aggregate_deg = _make_sc_aggregate(True)
_sc_aggregate = _make_sc_aggregate(False)


def _tc_layer_body(relu, in_split, out_split,
                   sref, dref, xref, wlref, bref, wrref, oref):
    agg = jnp.concatenate([sref[0], sref[1]], axis=-1)
    rec = 1.0 / jnp.maximum(dref[...], 1.0)
    agg = agg * rec
    if in_split:
        xx = jnp.concatenate([xref[0], xref[1]], axis=-1)
    else:
        xx = xref[...]
    o = (jnp.dot(agg, wlref[...], preferred_element_type=jnp.float32)
         + bref[...]
         + jnp.dot(xx, wrref[...], preferred_element_type=jnp.float32))
    if relu:
        o = jnp.maximum(o, 0.0)
    if out_split:
        oref[0] = o[:, :HD]
        oref[1] = o[:, HD:]
    else:
        oref[...] = o


def _tc_layer(summed, deg_col, xin, W_l, b, W_r, *, relu, in_split, out_split):
    BM = 2000
    grid = (N // BM,)
    split_spec = pl.BlockSpec((NC, BM, HD), lambda i: (0, i, 0))
    dense_spec = pl.BlockSpec((BM, D), lambda i: (i, 0))
    in_specs = [
        split_spec,
        pl.BlockSpec((BM, 1), lambda i: (i, 0)),
        split_spec if in_split else dense_spec,
        pl.BlockSpec((D, D), lambda i: (0, 0)),
        pl.BlockSpec((1, D), lambda i: (0, 0)),
        pl.BlockSpec((D, D), lambda i: (0, 0)),
    ]
    if out_split:
        out_spec = split_spec
        out_shape = jax.ShapeDtypeStruct((NC, N, HD), jnp.float32)
    else:
        out_spec = dense_spec
        out_shape = jax.ShapeDtypeStruct((N, D), jnp.float32)
    return pl.pallas_call(
        functools.partial(_tc_layer_body, relu, in_split, out_split),
        grid=grid,
        in_specs=in_specs,
        out_specs=out_spec,
        out_shape=out_shape,
    )(summed, deg_col, xin, W_l, b.reshape(1, D), W_r)


def kernel(x, edge_index, W1_l, b1, W1_r, W2_l, b2, W2_r):
    src = edge_index[0].astype(jnp.int32)
    dst = edge_index[1].astype(jnp.int32)
    sr = src.reshape(NS, PHASES, CPP, K)
    src_idx = jnp.stack([sr, sr + N])
    dst_idx = dst.reshape(NS, PHASES, CPP, K)
    x_flat = jnp.concatenate([x[:, :HD], x[:, HD:]], axis=0)

    summed1, deg_pad = _sc_aggregate_deg(x_flat, src_idx, dst_idx)
    deg_col = deg_pad[:N].reshape(N, 1)
    h_split = _tc_layer(summed1, deg_col, x, W1_l, b1, W1_r,
                        relu=True, in_split=False, out_split=True)
    summed2, _ = _sc_aggregate(h_split.reshape(NC * N, HD), src_idx, dst_idx)
    out = _tc_layer(summed2, deg_col, h_split, W2_l, b2, W2_r,
                    relu=False, in_split=True, out_split=False)
    return out
